# Initial kernel scaffold; baseline (speedup 1.0000x reference)
import jax
import jax.numpy as jnp
from jax.experimental import pallas as pl


def kernel(x, edge_index, W1, b1, W2, b2):
    raise NotImplementedError("write your pallas kernel here")


# speedup vs baseline: 11.8884x; 11.8884x over previous
import functools

import jax
import jax.numpy as jnp
from jax import lax
from jax.experimental import pallas as pl
from jax.experimental.pallas import tpu as pltpu
from jax.experimental.pallas import tpu_sc as plsc

N = 10000
E = 320000
D = 128

NC = 2
NS = 16
NW = NC * NS

K = 128
CH = (E + NW * K - 1) // (NW * K)
EPW = CH * K
EP = NW * EPW
NP = 10240
RPT = NP // NS

_MESH = plsc.VectorSubcoreMesh(core_axis_name="c", subcore_axis_name="s")


@functools.partial(
    pl.kernel,
    out_type=jax.ShapeDtypeStruct((NC, NP), jnp.float32),
    mesh=_MESH,
    scratch_types=[
        pltpu.VMEM((CH, K), jnp.int32),
        pltpu.VMEM((K,), jnp.float32),
        pltpu.VMEM_SHARED((NP,), jnp.float32),
    ],
)
def _sc_degree(dst3, ones_h, z1d, out, dst_v, ones_v, acc_sh):
    cid = lax.axis_index("c")
    sid = lax.axis_index("s")
    wid = cid * NS + sid
    row0 = pl.multiple_of(sid * RPT, RPT)
    pltpu.sync_copy(z1d, acc_sh.at[pl.ds(row0, RPT)])
    pltpu.sync_copy(dst3.at[wid], dst_v)
    pltpu.sync_copy(ones_h, ones_v)
    plsc.subcore_barrier()

    def body(j, carry):
        pltpu.sync_copy(ones_v, acc_sh.at[dst_v.at[j]], add=True)
        return carry

    lax.fori_loop(0, CH, body, 0)
    plsc.subcore_barrier()
    pltpu.sync_copy(acc_sh.at[pl.ds(row0, RPT)], out.at[cid, pl.ds(row0, RPT)])


@functools.partial(
    pl.kernel,
    out_type=jax.ShapeDtypeStruct((NC, NP, D), jnp.float32),
    mesh=_MESH,
    scratch_types=[
        pltpu.VMEM((CH, K), jnp.int32),
        pltpu.VMEM((CH, K), jnp.int32),
        pltpu.VMEM((K, D), jnp.float32),
        pltpu.VMEM_SHARED((NP, D), jnp.float32),
        pltpu.SemaphoreType.DMA,
    ],
)
def _sc_aggregate(table, src3, dst3, zblk, out, src_v, dst_v, rows_v, acc_sh,
                  gsem):
    cid = lax.axis_index("c")
    sid = lax.axis_index("s")
    wid = cid * NS + sid
    row0 = pl.multiple_of(sid * RPT, RPT)
    pltpu.sync_copy(zblk, acc_sh.at[pl.ds(row0, RPT)])
    pltpu.sync_copy(src3.at[wid], src_v)
    pltpu.sync_copy(dst3.at[wid], dst_v)
    plsc.subcore_barrier()

    def body(j, carry):
        pltpu.async_copy(table.at[src_v.at[j]], rows_v, gsem).wait()
        pltpu.sync_copy(rows_v, acc_sh.at[dst_v.at[j]], add=True)
        return carry

    lax.fori_loop(0, CH, body, 0)
    plsc.subcore_barrier()
    pltpu.sync_copy(acc_sh.at[pl.ds(row0, RPT)], out.at[cid, pl.ds(row0, RPT)])


_R = 2048


def _dinv_of(deg_ref):
    deg = deg_ref[:, 0:1] + deg_ref[:, 1:2] + 1.0
    return jnp.where(deg > 0, lax.rsqrt(deg), 0.0)


def _tc1_body(x_ref, w_ref, deg_ref, out_ref):
    dinv = _dinv_of(deg_ref)
    h = jnp.dot(x_ref[...], w_ref[...], preferred_element_type=jnp.float32)
    out_ref[...] = h * dinv


def _tc1(x_pad, W1, deg_t):
    return pl.pallas_call(
        _tc1_body,
        grid=(NP // _R,),
        in_specs=[
            pl.BlockSpec((_R, D), lambda i: (i, 0)),
            pl.BlockSpec((D, D), lambda i: (0, 0)),
            pl.BlockSpec((_R, NC), lambda i: (i, 0)),
        ],
        out_specs=pl.BlockSpec((_R, D), lambda i: (i, 0)),
        out_shape=jax.ShapeDtypeStruct((NP, D), jnp.float32),
    )(x_pad, W1, deg_t)


def _tc2_body(accp_ref, t1_ref, deg_ref, w_ref, b_ref, out_ref):
    dinv = _dinv_of(deg_ref)
    acc = accp_ref[0] + accp_ref[1]
    o1 = dinv * (acc + t1_ref[...]) + b_ref[...]
    g = jnp.maximum(o1, 0.0)
    h2 = jnp.dot(g, w_ref[...], preferred_element_type=jnp.float32)
    out_ref[...] = h2 * dinv


def _tc2(acc1, table1, deg_t, W2, b1):
    return pl.pallas_call(
        _tc2_body,
        grid=(NP // _R,),
        in_specs=[
            pl.BlockSpec((NC, _R, D), lambda i: (0, i, 0)),
            pl.BlockSpec((_R, D), lambda i: (i, 0)),
            pl.BlockSpec((_R, NC), lambda i: (i, 0)),
            pl.BlockSpec((D, D), lambda i: (0, 0)),
            pl.BlockSpec((1, D), lambda i: (0, 0)),
        ],
        out_specs=pl.BlockSpec((_R, D), lambda i: (i, 0)),
        out_shape=jax.ShapeDtypeStruct((NP, D), jnp.float32),
    )(acc1, table1, deg_t, W2, b1)


def _tc3_body(accp_ref, t2_ref, deg_ref, b_ref, out_ref):
    dinv = _dinv_of(deg_ref)
    acc = accp_ref[0] + accp_ref[1]
    out_ref[...] = dinv * (acc + t2_ref[...]) + b_ref[...]


def _tc3(acc2, table2, deg_t, b2):
    return pl.pallas_call(
        _tc3_body,
        grid=(NP // _R,),
        in_specs=[
            pl.BlockSpec((NC, _R, D), lambda i: (0, i, 0)),
            pl.BlockSpec((_R, D), lambda i: (i, 0)),
            pl.BlockSpec((_R, NC), lambda i: (i, 0)),
            pl.BlockSpec((1, D), lambda i: (0, 0)),
        ],
        out_specs=pl.BlockSpec((_R, D), lambda i: (i, 0)),
        out_shape=jax.ShapeDtypeStruct((NP, D), jnp.float32),
    )(acc2, table2, deg_t, b2)


def kernel(x, edge_index, W1, b1, W2, b2):
    pad = EP - E
    padv = jnp.full((pad,), N, dtype=jnp.int32)
    src3 = jnp.concatenate([edge_index[0], padv]).reshape(NW, CH, K)
    dst3 = jnp.concatenate([edge_index[1], padv]).reshape(NW, CH, K)
    x_pad = jnp.concatenate(
        [x, jnp.zeros((NP - N, D), dtype=jnp.float32)], axis=0)
    zblk = jnp.zeros((RPT, D), dtype=jnp.float32)
    z1d = jnp.zeros((RPT,), dtype=jnp.float32)
    ones_h = jnp.ones((K,), dtype=jnp.float32)

    degp = _sc_degree(dst3, ones_h, z1d)
    deg_t = degp.T

    table1 = _tc1(x_pad, W1, deg_t)
    acc1 = _sc_aggregate(table1, src3, dst3, zblk)
    table2 = _tc2(acc1, table1, deg_t, W2, b1.reshape(1, D))
    acc2 = _sc_aggregate(table2, src3, dst3, zblk)
    out = _tc3(acc2, table2, deg_t, b2.reshape(1, D))
    return out[:N]
